# Initial kernel scaffold; baseline (speedup 1.0000x reference)
import jax
import jax.numpy as jnp
from jax.experimental import pallas as pl


def kernel(x, table):
    raise NotImplementedError("write your pallas kernel here")


# speedup vs baseline: 4.5672x; 4.5672x over previous
import functools

import jax
import jax.numpy as jnp
from jax import lax
from jax.experimental import pallas as pl
from jax.experimental.pallas import tpu as pltpu
from jax.experimental.pallas import tpu_sc as plsc

NUM_CORES = 2
NUM_SUBCORES = 16
NUM_WORKERS = NUM_CORES * NUM_SUBCORES
TOTAL_ROWS = 4096 * 50
ROWS_PER_WORKER = TOTAL_ROWS // NUM_WORKERS
IDX_MINOR = 128
IDX_ROWS = ROWS_PER_WORKER // IDX_MINOR
CHUNK_GATHERS = 5
CHUNK_ROWS = CHUNK_GATHERS * IDX_MINOR
NUM_CHUNKS = ROWS_PER_WORKER // CHUNK_ROWS
VREGS_PER_CHUNK = CHUNK_ROWS // 16
D = 64
DUMP_ROW = CHUNK_ROWS


@functools.partial(
    pl.kernel,
    out_type=jax.ShapeDtypeStruct((TOTAL_ROWS, D), jnp.float32),
    mesh=plsc.VectorSubcoreMesh(core_axis_name="c", subcore_axis_name="s"),
    compiler_params=pltpu.CompilerParams(use_tc_tiling_on_sc=False),
    scratch_types=[
        pltpu.VMEM((IDX_ROWS, IDX_MINOR), jnp.int32),
        pltpu.VMEM((CHUNK_ROWS + 8, D), jnp.float32),
        pltpu.SemaphoreType.DMA,
    ],
)
def _lookup(x_hbm, table_hbm, out_hbm, idx_v, rows_v, sem):
    wid = lax.axis_index("s") * NUM_CORES + lax.axis_index("c")
    base = wid * ROWS_PER_WORKER
    pltpu.sync_copy(x_hbm.at[wid], idx_v)

    def chunk_body(c, _):
        copies = []
        for j in range(CHUNK_GATHERS):
            cp = pltpu.async_copy(
                table_hbm.at[idx_v.at[c * CHUNK_GATHERS + j]],
                rows_v.at[pl.ds(j * IDX_MINOR, IDX_MINOR)],
                sem,
            )
            copies.append(cp)

        def min_body(v, acc):
            row = c * CHUNK_GATHERS + v // 8
            col = (v % 8) * 16
            return jnp.minimum(acc, idx_v[row, pl.ds(col, 16)])

        acc = lax.fori_loop(
            0, VREGS_PER_CHUNK, min_body, jnp.full((16,), 1, jnp.int32)
        )
        mn = acc[0]
        for i in range(1, 16):
            mn = jnp.minimum(mn, acc[i])

        for cp in copies:
            cp.wait()

        @pl.when(mn == 0)
        def _():
            def fix_body(v, _):
                row = c * CHUNK_GATHERS + v // 8
                col = (v % 8) * 16
                vals = idx_v[row, pl.ds(col, 16)]
                zeros16 = jnp.zeros((16,), jnp.float32)
                for lane in range(16):
                    @pl.when(vals[lane] == 0)
                    def _():
                        r = v * 16 + lane
                        rows_v[r, pl.ds(0, 16)] = zeros16
                        rows_v[r, pl.ds(16, 16)] = zeros16
                        rows_v[r, pl.ds(32, 16)] = zeros16
                        rows_v[r, pl.ds(48, 16)] = zeros16

                return 0

            lax.fori_loop(0, VREGS_PER_CHUNK, fix_body, 0)

        pltpu.sync_copy(
            rows_v.at[pl.ds(0, CHUNK_ROWS)],
            out_hbm.at[pl.ds(base + c * CHUNK_ROWS, CHUNK_ROWS)],
        )
        return 0

    lax.fori_loop(0, NUM_CHUNKS, chunk_body, 0)


def kernel(x, table):
    x_flat = x.astype(jnp.int32).reshape(NUM_WORKERS, IDX_ROWS, IDX_MINOR)
    out = _lookup(x_flat, table)
    return out.reshape(4096, 50, D)


# speedup vs baseline: 4.6228x; 1.0122x over previous
import functools

import jax
import jax.numpy as jnp
from jax import lax
from jax.experimental import pallas as pl
from jax.experimental.pallas import tpu as pltpu
from jax.experimental.pallas import tpu_sc as plsc

NUM_CORES = 2
NUM_SUBCORES = 16
NUM_WORKERS = NUM_CORES * NUM_SUBCORES
TOTAL_ROWS = 4096 * 50
ROWS_PER_WORKER = TOTAL_ROWS // NUM_WORKERS
IDX_MINOR = 128
IDX_ROWS = ROWS_PER_WORKER // IDX_MINOR
CHUNK_GATHERS = 5
CHUNK_ROWS = CHUNK_GATHERS * IDX_MINOR
NUM_CHUNKS = ROWS_PER_WORKER // CHUNK_ROWS
VREGS_PER_CHUNK = CHUNK_ROWS // 16
D = 64


@functools.partial(
    pl.kernel,
    out_type=jax.ShapeDtypeStruct((TOTAL_ROWS, D), jnp.float32),
    mesh=plsc.VectorSubcoreMesh(core_axis_name="c", subcore_axis_name="s"),
    compiler_params=pltpu.CompilerParams(use_tc_tiling_on_sc=False),
    scratch_types=[
        pltpu.VMEM((IDX_ROWS, IDX_MINOR), jnp.int32),
        pltpu.VMEM((CHUNK_ROWS, D), jnp.float32),
        pltpu.VMEM((CHUNK_ROWS, D), jnp.float32),
        pltpu.SemaphoreType.DMA,
        pltpu.SemaphoreType.DMA,
        pltpu.SemaphoreType.DMA,
        pltpu.SemaphoreType.DMA,
    ],
)
def _lookup(x_hbm, table_hbm, out_hbm, idx_v, rows_a, rows_b,
            sem_ga, sem_gb, sem_wa, sem_wb):
    wid = lax.axis_index("s") * NUM_CORES + lax.axis_index("c")
    base = wid * ROWS_PER_WORKER
    pltpu.sync_copy(x_hbm.at[wid], idx_v)

    bufs = (rows_a, rows_b)
    gsems = (sem_ga, sem_gb)
    wsems = (sem_wa, sem_wb)

    def fire(c, buf, gsem):
        return [
            pltpu.async_copy(
                table_hbm.at[idx_v.at[c * CHUNK_GATHERS + j]],
                buf.at[pl.ds(j * IDX_MINOR, IDX_MINOR)],
                gsem,
            )
            for j in range(CHUNK_GATHERS)
        ]

    def detect(c):
        def min_body(v, acc):
            row = c * CHUNK_GATHERS + v // 8
            col = (v % 8) * 16
            return jnp.minimum(acc, idx_v[row, pl.ds(col, 16)])

        acc = lax.fori_loop(
            0, VREGS_PER_CHUNK, min_body, jnp.full((16,), 1, jnp.int32)
        )
        mn = acc[0]
        for i in range(1, 16):
            mn = jnp.minimum(mn, acc[i])
        return mn

    def fix(c, buf):
        def fix_body(v, _):
            row = c * CHUNK_GATHERS + v // 8
            col = (v % 8) * 16
            vals = idx_v[row, pl.ds(col, 16)]
            zeros16 = jnp.zeros((16,), jnp.float32)
            for lane in range(16):
                @pl.when(vals[lane] == 0)
                def _():
                    r = v * 16 + lane
                    buf[r, pl.ds(0, 16)] = zeros16
                    buf[r, pl.ds(16, 16)] = zeros16
                    buf[r, pl.ds(32, 16)] = zeros16
                    buf[r, pl.ds(48, 16)] = zeros16

            return 0

        lax.fori_loop(0, VREGS_PER_CHUNK, fix_body, 0)

    gathers = {0: fire(0, bufs[0], gsems[0])}
    writebacks = {}
    for c in range(NUM_CHUNKS):
        p = c % 2
        buf, gsem, wsem = bufs[p], gsems[p], wsems[p]
        if c + 1 < NUM_CHUNKS:
            if c - 1 >= 0:
                writebacks.pop(c - 1).wait()
            gathers[c + 1] = fire(c + 1, bufs[1 - p], gsems[1 - p])
        mn = detect(c)
        for cp in gathers.pop(c):
            cp.wait()
        pl.when(mn == 0)(lambda c=c, buf=buf: fix(c, buf))
        writebacks[c] = pltpu.async_copy(
            buf, out_hbm.at[pl.ds(base + c * CHUNK_ROWS, CHUNK_ROWS)], wsem
        )
    writebacks.pop(NUM_CHUNKS - 2).wait()
    writebacks.pop(NUM_CHUNKS - 1).wait()


def kernel(x, table):
    x_flat = x.astype(jnp.int32).reshape(NUM_WORKERS, IDX_ROWS, IDX_MINOR)
    out = _lookup(x_flat, table)
    return out.reshape(4096, 50, D)


# speedup vs baseline: 4.6273x; 1.0010x over previous
import functools

import jax
import jax.numpy as jnp
from jax import lax
from jax.experimental import pallas as pl
from jax.experimental.pallas import tpu as pltpu
from jax.experimental.pallas import tpu_sc as plsc

NUM_CORES = 2
NUM_SUBCORES = 16
NUM_WORKERS = NUM_CORES * NUM_SUBCORES
TOTAL_ROWS = 4096 * 50
ROWS_PER_WORKER = TOTAL_ROWS // NUM_WORKERS
CHUNK_ROWS = 640
NUM_CHUNKS = ROWS_PER_WORKER // CHUNK_ROWS
VREGS_PER_CHUNK = CHUNK_ROWS // 16
D = 64


@functools.partial(
    pl.kernel,
    out_type=jax.ShapeDtypeStruct((TOTAL_ROWS, D), jnp.float32),
    mesh=plsc.VectorSubcoreMesh(core_axis_name="c", subcore_axis_name="s"),
    compiler_params=pltpu.CompilerParams(use_tc_tiling_on_sc=False),
    scratch_types=[
        pltpu.VMEM((NUM_CHUNKS, CHUNK_ROWS), jnp.int32),
        pltpu.VMEM((CHUNK_ROWS, D), jnp.float32),
        pltpu.VMEM((CHUNK_ROWS, D), jnp.float32),
        pltpu.SemaphoreType.DMA,
        pltpu.SemaphoreType.DMA,
        pltpu.SemaphoreType.DMA,
        pltpu.SemaphoreType.DMA,
    ],
)
def _lookup(x_hbm, table_hbm, out_hbm, idx_v, rows_a, rows_b,
            sem_ga, sem_gb, sem_wa, sem_wb):
    wid = lax.axis_index("s") * NUM_CORES + lax.axis_index("c")
    base = wid * ROWS_PER_WORKER
    pltpu.sync_copy(x_hbm.at[wid], idx_v)

    bufs = (rows_a, rows_b)
    gsems = (sem_ga, sem_gb)
    wsems = (sem_wa, sem_wb)

    def fire(c, buf, gsem):
        return [pltpu.async_copy(table_hbm.at[idx_v.at[c]], buf, gsem)]

    def detect(c):
        def min_body(v, acc):
            return jnp.minimum(acc, idx_v[c, pl.ds(v * 16, 16)])

        acc = lax.fori_loop(
            0, VREGS_PER_CHUNK, min_body, jnp.full((16,), 1, jnp.int32)
        )
        mn = acc[0]
        for i in range(1, 16):
            mn = jnp.minimum(mn, acc[i])
        return mn

    def fix(c, buf):
        def fix_body(v, _):
            vals = idx_v[c, pl.ds(v * 16, 16)]
            zeros16 = jnp.zeros((16,), jnp.float32)
            for lane in range(16):
                @pl.when(vals[lane] == 0)
                def _():
                    r = v * 16 + lane
                    buf[r, pl.ds(0, 16)] = zeros16
                    buf[r, pl.ds(16, 16)] = zeros16
                    buf[r, pl.ds(32, 16)] = zeros16
                    buf[r, pl.ds(48, 16)] = zeros16

            return 0

        lax.fori_loop(0, VREGS_PER_CHUNK, fix_body, 0)

    gathers = {0: fire(0, bufs[0], gsems[0])}
    writebacks = {}
    for c in range(NUM_CHUNKS):
        p = c % 2
        buf, gsem, wsem = bufs[p], gsems[p], wsems[p]
        if c + 1 < NUM_CHUNKS:
            if c - 1 >= 0:
                writebacks.pop(c - 1).wait()
            gathers[c + 1] = fire(c + 1, bufs[1 - p], gsems[1 - p])
        mn = detect(c)
        for cp in gathers.pop(c):
            cp.wait()
        pl.when(mn == 0)(lambda c=c, buf=buf: fix(c, buf))
        writebacks[c] = pltpu.async_copy(
            buf, out_hbm.at[pl.ds(base + c * CHUNK_ROWS, CHUNK_ROWS)], wsem
        )
    writebacks.pop(NUM_CHUNKS - 2).wait()
    writebacks.pop(NUM_CHUNKS - 1).wait()


def kernel(x, table):
    x_flat = x.astype(jnp.int32).reshape(NUM_WORKERS, NUM_CHUNKS, CHUNK_ROWS)
    out = _lookup(x_flat, table)
    return out.reshape(4096, 50, D)
